# baseline (speedup 1.0000x reference)
import functools

import jax
import jax.numpy as jnp
from jax import lax
from jax.experimental import pallas as pl
from jax.experimental.pallas import tpu as pltpu
from jax.experimental.pallas import tpu_sc as plsc

_INFO = plsc.get_sparse_core_info()
_NC = _INFO.num_cores
_NS = _INFO.num_subcores
_NW = _NC * _NS
_L = 16

_SB = 128
_LOOK = 2


def _make_kernel(vocab, dim, n_seq, seq_len):
    assert n_seq == _SB * _NW and seq_len % _LOOK == 0 and dim % _L == 0
    mesh = plsc.VectorSubcoreMesh(core_axis_name="c", subcore_axis_name="s")

    @functools.partial(
        pl.kernel,
        mesh=mesh,
        compiler_params=pltpu.CompilerParams(needs_layout_passes=False),
        out_type=jax.ShapeDtypeStruct((seq_len, dim, n_seq), jnp.float32),
        scratch_types=[
            pltpu.VMEM((seq_len, _SB), jnp.int32),
            pltpu.VMEM((_LOOK, _SB, 2 * dim), jnp.float32),
            pltpu.VMEM((_LOOK, dim, _SB), jnp.float32),
            [pltpu.SemaphoreType.DMA] * _LOOK,
            [pltpu.SemaphoreType.DMA] * _LOOK,
        ],
    )
    def gather_kernel(table_hbm, idx_hbm, out_hbm, idx_v, gbuf, tbuf, gsem,
                      ssem):
        wid = lax.axis_index("s") * _NC + lax.axis_index("c")

        pltpu.sync_copy(idx_hbm.at[wid], idx_v)

        lane = lax.iota(jnp.int32, _L)
        rows = [lane + g * _L for g in range(_SB // _L)]

        def gather_descr(p, b):
            return pltpu.make_async_copy(
                table_hbm.at[idx_v.at[p]], gbuf.at[b], gsem[b]
            )

        def store_descr(p, b):
            return pltpu.make_async_copy(
                tbuf.at[b],
                out_hbm.at[p].at[:, pl.ds(wid * _SB, _SB)],
                ssem[b],
            )

        def transpose(b):
            gb = gbuf.at[b]
            tb = tbuf.at[b]
            for e in range(dim):
                col = jnp.full_like(lane, e)
                for g in range(_SB // _L):
                    tb[e, pl.ds(g * _L, _L)] = plsc.load_gather(
                        gb, [rows[g], col]
                    )

        def step(p, b):
            gather_descr(p, b).wait()

            @pl.when(p >= _LOOK)
            def _():
                store_descr(p - _LOOK, b).wait()

            transpose(b)
            store_descr(p, b).start()

            @pl.when(p + _LOOK < seq_len)
            def _():
                gather_descr(p + _LOOK, b).start()

        for b in range(_LOOK):
            gather_descr(b, b).start()

        def body(g, c):
            for u in range(_LOOK):
                step(g * _LOOK + u, u)
            return c

        lax.fori_loop(0, seq_len // _LOOK, body, 0)
        for b in range(_LOOK):
            store_descr(seq_len - _LOOK + b, b).wait()

    return gather_kernel


def kernel(t, table):
    vocab, dim = table.shape
    n_seq, seq_len = t.shape
    table_pad = jnp.pad(table, ((0, 0), (0, 128 - dim)))
    idx = t.reshape(_NW, _SB, seq_len).transpose(0, 2, 1).astype(jnp.int32)
    out3 = _make_kernel(vocab, dim, n_seq, seq_len)(table_pad, idx)
    return out3.transpose(2, 0, 1)


# speedup vs baseline: 1.4232x; 1.4232x over previous
import functools

import jax
import jax.numpy as jnp
from jax import lax
from jax.experimental import pallas as pl
from jax.experimental.pallas import tpu as pltpu
from jax.experimental.pallas import tpu_sc as plsc

_INFO = plsc.get_sparse_core_info()
_NC = _INFO.num_cores
_NS = _INFO.num_subcores
_NW = _NC * _NS
_L = 16

_SB = 128
_LOOK = 2


def _make_kernel(vocab, dim, n_seq, seq_len):
    assert n_seq == _SB * _NW and seq_len % _LOOK == 0 and dim % _L == 0
    mesh = plsc.VectorSubcoreMesh(core_axis_name="c", subcore_axis_name="s")

    @functools.partial(
        pl.kernel,
        mesh=mesh,
        compiler_params=pltpu.CompilerParams(needs_layout_passes=False),
        out_type=jax.ShapeDtypeStruct((seq_len, dim, n_seq), jnp.float32),
        scratch_types=[
            pltpu.VMEM((seq_len, _SB), jnp.int32),
            pltpu.VMEM((_LOOK, _SB, 2 * dim), jnp.float32),
            pltpu.VMEM((_LOOK, dim, _SB), jnp.float32),
            [pltpu.SemaphoreType.DMA] * _LOOK,
            [pltpu.SemaphoreType.DMA] * _LOOK,
        ],
    )
    def gather_kernel(table_hbm, idx_hbm, out_hbm, idx_v, gbuf, tbuf, gsem,
                      ssem):
        wid = lax.axis_index("s") * _NC + lax.axis_index("c")

        pltpu.sync_copy(idx_hbm.at[wid], idx_v)

        lane = lax.iota(jnp.int32, _L)
        rows = [lane + g * _L for g in range(_SB // _L)]

        def gather_descr(p, b):
            return pltpu.make_async_copy(
                table_hbm.at[idx_v.at[p]], gbuf.at[b], gsem[b]
            )

        def store_descr(p, b):
            return pltpu.make_async_copy(
                tbuf.at[b],
                out_hbm.at[p].at[:, pl.ds(wid * _SB, _SB)],
                ssem[b],
            )

        zero_vec = lax.shift_right_logical(idx_v[0, pl.ds(0, _L)], 31)

        def transpose(b):
            gb = gbuf.at[b]
            tb = tbuf.at[b]
            for e in range(dim):
                col = zero_vec + e
                vals = [
                    plsc.load_gather(gb, [rows[g], col])
                    for g in range(_SB // _L)
                ]
                for g in range(_SB // _L):
                    tb[e, pl.ds(g * _L, _L)] = vals[g]

        def step(p, b):
            gather_descr(p, b).wait()

            @pl.when(p >= _LOOK)
            def _():
                store_descr(p - _LOOK, b).wait()

            transpose(b)
            store_descr(p, b).start()

            @pl.when(p + _LOOK < seq_len)
            def _():
                gather_descr(p + _LOOK, b).start()

        for b in range(_LOOK):
            gather_descr(b, b).start()

        def body(g, c):
            for u in range(_LOOK):
                step(g * _LOOK + u, u)
            return c

        lax.fori_loop(0, seq_len // _LOOK, body, 0)
        for b in range(_LOOK):
            store_descr(seq_len - _LOOK + b, b).wait()

    return gather_kernel


def kernel(t, table):
    vocab, dim = table.shape
    n_seq, seq_len = t.shape
    table_pad = jnp.pad(table, ((0, 0), (0, 128 - dim)))
    idx = t.reshape(_NW, _SB, seq_len).transpose(0, 2, 1).astype(jnp.int32)
    out3 = _make_kernel(vocab, dim, n_seq, seq_len)(table_pad, idx)
    return out3.transpose(2, 0, 1)


# speedup vs baseline: 3.6340x; 2.5533x over previous
import functools

import jax
import jax.numpy as jnp
from jax import lax
from jax.experimental import pallas as pl
from jax.experimental.pallas import tpu as pltpu
from jax.experimental.pallas import tpu_sc as plsc

_INFO = plsc.get_sparse_core_info()
_NC = _INFO.num_cores
_NS = _INFO.num_subcores
_NW = _NC * _NS
_L = 16

_SB = 128
_LOOK = 2


def _make_kernel(vocab, dim, n_seq, seq_len):
    assert n_seq == _SB * _NW and seq_len % _LOOK == 0 and dim % _L == 0
    mesh = plsc.VectorSubcoreMesh(core_axis_name="c", subcore_axis_name="s")

    @functools.partial(
        pl.kernel,
        mesh=mesh,
        compiler_params=pltpu.CompilerParams(needs_layout_passes=False),
        out_type=jax.ShapeDtypeStruct((seq_len, dim, n_seq), jnp.float32),
        scratch_types=[
            pltpu.VMEM((seq_len, _SB), jnp.int32),
            pltpu.VMEM((_LOOK, _SB, 2 * dim), jnp.float32),
            pltpu.VMEM((_LOOK, dim, _SB), jnp.float32),
            [pltpu.SemaphoreType.DMA] * _LOOK,
            [pltpu.SemaphoreType.DMA] * _LOOK,
        ],
    )
    def gather_kernel(table_hbm, idx_hbm, out_hbm, idx_v, gbuf, tbuf, gsem,
                      ssem):
        wid = lax.axis_index("s") * _NC + lax.axis_index("c")

        pltpu.sync_copy(idx_hbm.at[wid], idx_v)

        lane = lax.iota(jnp.int32, _L)
        rot = [(lane + d) % _L for d in range(_L)]

        def gather_descr(p, b):
            return pltpu.make_async_copy(
                table_hbm.at[idx_v.at[p]], gbuf.at[b], gsem[b]
            )

        def store_descr(p, b):
            return pltpu.make_async_copy(
                tbuf.at[b],
                out_hbm.at[p].at[:, pl.ds(wid * _SB, _SB)],
                ssem[b],
            )

        zero_s = jnp.sum(lax.shift_right_logical(idx_v[0, pl.ds(0, _L)], 31))

        def transpose(b):
            gb = gbuf.at[b]
            tb = tbuf.at[b]
            for g in range(_SB // _L):
                gblk = gb.at[pl.ds(g * _L, _L)]
                scol = zero_s + g * _L
                for eb in range(dim // _L):
                    tblk = tb.at[pl.ds(eb * _L, _L)]
                    ecol = zero_s + eb * _L
                    for d0 in range(0, _L, 8):
                        vals = [
                            plsc.load_gather(gblk, [lane, rot[d] + ecol])
                            for d in range(d0, d0 + 8)
                        ]
                        for i, d in enumerate(range(d0, d0 + 8)):
                            plsc.store_scatter(
                                tblk, [rot[d], lane + scol], vals[i]
                            )

        def step(p, b):
            gather_descr(p, b).wait()

            @pl.when(p >= _LOOK)
            def _():
                store_descr(p - _LOOK, b).wait()

            transpose(b)
            store_descr(p, b).start()

            @pl.when(p + _LOOK < seq_len)
            def _():
                gather_descr(p + _LOOK, b).start()

        for b in range(_LOOK):
            gather_descr(b, b).start()

        def body(g, c):
            for u in range(_LOOK):
                step(g * _LOOK + u, u)
            return c

        lax.fori_loop(0, seq_len // _LOOK, body, 0)
        for b in range(_LOOK):
            store_descr(seq_len - _LOOK + b, b).wait()

    return gather_kernel


def kernel(t, table):
    vocab, dim = table.shape
    n_seq, seq_len = t.shape
    table_pad = jnp.pad(table, ((0, 0), (0, 128 - dim)))
    idx = t.reshape(_NW, _SB, seq_len).transpose(0, 2, 1).astype(jnp.int32)
    out3 = _make_kernel(vocab, dim, n_seq, seq_len)(table_pad, idx)
    return out3.transpose(2, 0, 1)


# speedup vs baseline: 3.6748x; 1.0112x over previous
import functools

import jax
import jax.numpy as jnp
from jax import lax
from jax.experimental import pallas as pl
from jax.experimental.pallas import tpu as pltpu
from jax.experimental.pallas import tpu_sc as plsc

_INFO = plsc.get_sparse_core_info()
_NC = _INFO.num_cores
_NS = _INFO.num_subcores
_NW = _NC * _NS
_L = 16

_SB = 128
_GBUF = 4
_TBUF = 2


def _make_kernel(vocab, dim, n_seq, seq_len):
    assert n_seq == _SB * _NW and seq_len % _GBUF == 0 and dim % _L == 0
    mesh = plsc.VectorSubcoreMesh(core_axis_name="c", subcore_axis_name="s")

    @functools.partial(
        pl.kernel,
        mesh=mesh,
        compiler_params=pltpu.CompilerParams(needs_layout_passes=False),
        out_type=jax.ShapeDtypeStruct((seq_len, dim, n_seq), jnp.float32),
        scratch_types=[
            pltpu.VMEM((seq_len, _SB), jnp.int32),
            pltpu.VMEM((_GBUF, _SB, 2 * dim), jnp.float32),
            pltpu.VMEM((_TBUF, dim, _SB), jnp.float32),
            [pltpu.SemaphoreType.DMA] * _GBUF,
            [pltpu.SemaphoreType.DMA] * _TBUF,
        ],
    )
    def gather_kernel(table_hbm, idx_hbm, out_hbm, idx_v, gbuf, tbuf, gsem,
                      ssem):
        wid = lax.axis_index("s") * _NC + lax.axis_index("c")

        pltpu.sync_copy(idx_hbm.at[wid], idx_v)

        lane = lax.iota(jnp.int32, _L)
        rot = [(lane + d) % _L for d in range(_L)]

        def gather_descr(p, b):
            return pltpu.make_async_copy(
                table_hbm.at[idx_v.at[p]], gbuf.at[b], gsem[b]
            )

        def store_descr(p, b):
            return pltpu.make_async_copy(
                tbuf.at[b],
                out_hbm.at[p].at[:, pl.ds(wid * _SB, _SB)],
                ssem[b],
            )

        zero_s = jnp.sum(lax.shift_right_logical(idx_v[0, pl.ds(0, _L)], 31))

        def transpose(b, tslot):
            gb = gbuf.at[b]
            tb = tbuf.at[tslot]
            for g in range(_SB // _L):
                gblk = gb.at[pl.ds(g * _L, _L)]
                scol = zero_s + g * _L
                for eb in range(dim // _L):
                    tblk = tb.at[pl.ds(eb * _L, _L)]
                    ecol = zero_s + eb * _L
                    for d0 in range(0, _L, 8):
                        vals = [
                            plsc.load_gather(gblk, [lane, rot[d] + ecol])
                            for d in range(d0, d0 + 8)
                        ]
                        for i, d in enumerate(range(d0, d0 + 8)):
                            plsc.store_scatter(
                                tblk, [rot[d], lane + scol], vals[i]
                            )

        def step(p, gb_slot, tb_slot):
            gather_descr(p, gb_slot).wait()

            @pl.when(p >= _TBUF)
            def _():
                store_descr(p - _TBUF, tb_slot).wait()

            transpose(gb_slot, tb_slot)
            store_descr(p, tb_slot).start()

            @pl.when(p + _GBUF < seq_len)
            def _():
                gather_descr(p + _GBUF, gb_slot).start()

        for b in range(_GBUF):
            gather_descr(b, b).start()

        def body(g, c):
            for u in range(_GBUF):
                p = g * _GBUF + u
                step(p, u, u % _TBUF)
            return c

        lax.fori_loop(0, seq_len // _GBUF, body, 0)
        for b in range(_TBUF):
            store_descr(seq_len - _TBUF + b, b).wait()

    return gather_kernel


def kernel(t, table):
    vocab, dim = table.shape
    n_seq, seq_len = t.shape
    table_pad = jnp.pad(table, ((0, 0), (0, 128 - dim)))
    idx = t.reshape(_NW, _SB, seq_len).transpose(0, 2, 1).astype(jnp.int32)
    out3 = _make_kernel(vocab, dim, n_seq, seq_len)(table_pad, idx)
    return out3.transpose(2, 0, 1)
